# baseline (speedup 1.0000x reference)
import functools

import jax
import jax.numpy as jnp
from jax import lax
from jax.experimental import pallas as pl
from jax.experimental.pallas import tpu as pltpu
from jax.experimental.pallas import tpu_sc as plsc

_B = 1024
_V = 100000
_K = 5
_HALF = _B // 2

_R_TC = 512
_R_SC = _B - _R_TC

_RB = 16

_NC = 2
_NS = 16
_NW = _NC * _NS
_LANES = 16
_CHUNK = 20000
_NCHUNK = _V // _CHUNK
_RPW = _R_SC // _NW


def _tc_sumexp_body(x_ref, s_ref):
    s_ref[...] = jnp.sum(jnp.exp(x_ref[...]), axis=1, keepdims=True)


def _tc_sumexp(x):
    return pl.pallas_call(
        _tc_sumexp_body,
        grid=(_R_TC // _RB,),
        in_specs=[pl.BlockSpec((_RB, _V), lambda r: (r, 0))],
        out_specs=pl.BlockSpec((_RB, 1), lambda r: (r, 0)),
        out_shape=jax.ShapeDtypeStruct((_R_TC, 1), jnp.float32),
    )(x)


_sc_mesh = plsc.VectorSubcoreMesh(core_axis_name="c", subcore_axis_name="s")


@functools.partial(
    pl.kernel,
    mesh=_sc_mesh,
    out_type=jax.ShapeDtypeStruct((_R_SC * _LANES,), jnp.float32),
    scratch_types=[
        pltpu.VMEM((_CHUNK,), jnp.float32),
        pltpu.VMEM((_CHUNK,), jnp.float32),
        pltpu.VMEM((_LANES,), jnp.float32),
        pltpu.SemaphoreType.DMA,
        pltpu.SemaphoreType.DMA,
    ],
)
def _sc_sumexp(xflat_hbm, out_hbm, buf0, buf1, rowbuf, sem0, sem1):
    wid = lax.axis_index("s") * _NC + lax.axis_index("c")
    row0 = _R_TC + wid * _RPW
    obase = wid * _RPW
    bufs = (buf0, buf1)
    sems = (sem0, sem1)

    def row_body(j, carry):
        row = row0 + j
        cps = [
            pltpu.make_async_copy(
                xflat_hbm.at[pl.ds(row * _V + c * _CHUNK, _CHUNK)],
                bufs[c % 2],
                sems[c % 2],
            )
            for c in range(_NCHUNK)
        ]
        cps[0].start()
        acc = jnp.zeros((_LANES,), jnp.float32)
        for c in range(_NCHUNK):
            if c + 1 < _NCHUNK:
                cps[c + 1].start()
            cps[c].wait()
            buf = bufs[c % 2]

            def vbody(i, a):
                off = pl.multiple_of(i * _LANES, _LANES)
                return a + jnp.exp(buf[pl.ds(off, _LANES)])

            acc = lax.fori_loop(0, _CHUNK // _LANES, vbody, acc, unroll=8)
        rowbuf[...] = acc
        pltpu.sync_copy(rowbuf, out_hbm.at[pl.ds((obase + j) * _LANES, _LANES)])
        return carry

    lax.fori_loop(0, _RPW, row_body, 0)


def _combine_body(g1_ref, g2_ref, gn_ref, anc_ref, s1_ref, p2_ref,
                  inst_ref, ans_ref):
    s1 = s1_ref[...]
    s2 = jnp.sum(p2_ref[...], axis=1, keepdims=True)
    p1 = jnp.exp(g1_ref[...]) / s1
    p2 = jnp.exp(g2_ref[...]) / s2
    pn = jnp.sum(jnp.exp(gn_ref[...]), axis=1, keepdims=True) / s1
    a = anc_ref[...]
    ans = -jnp.log(p1 + p2 + pn)
    inst = -jnp.log(p1 + p2)
    ans_ref[...] = (jnp.sum(a * ans) / _B).reshape(1, 1)
    inst_ref[...] = (jnp.sum((1.0 - a) * inst) / _B).reshape(1, 1)


def _combine(g1, g2, gn, anc, s1, partials2):
    return pl.pallas_call(
        _combine_body,
        out_shape=(
            jax.ShapeDtypeStruct((1, 1), jnp.float32),
            jax.ShapeDtypeStruct((1, 1), jnp.float32),
        ),
    )(g1, g2, gn, anc, s1, partials2)


def kernel(x, y, position, neighbours):
    s1 = _tc_sumexp(x)
    partials2 = _sc_sumexp(x.reshape(-1)).reshape(_R_SC, _LANES)

    y1 = y[:_HALF]
    y2 = y[_HALF:]
    pos = position[y1]
    anchor = (pos >= 0).astype(jnp.float32).reshape(_HALF, 1)
    pc = jnp.maximum(pos, 0)
    ncols = neighbours[pc]
    rows = jnp.arange(_HALF)
    g1 = x[rows, y1].reshape(_HALF, 1)
    g2 = x[rows + _HALF, y2].reshape(_HALF, 1)
    gn = x[rows[:, None], ncols]

    inst, ans = _combine(g1, g2, gn, anchor, s1, partials2)
    return (inst[0, 0], ans[0, 0])


# speedup vs baseline: 2.1256x; 2.1256x over previous
import functools

import jax
import jax.numpy as jnp
from jax import lax
from jax.experimental import pallas as pl
from jax.experimental.pallas import tpu as pltpu
from jax.experimental.pallas import tpu_sc as plsc

_B = 1024
_V = 100000
_K = 5
_HALF = _B // 2

_R_TC = 512
_R_SC = _B - _R_TC

_RB = 16

_NC = 2
_NS = 16
_NW = _NC * _NS
_LANES = 16
_CW = 3200
_NFULL = _V // _CW
_CTAIL = _V - _NFULL * _CW
_GPWK = _R_SC // (8 * _NW)


def _tc_sumexp_body(x_ref, s_ref):
    s_ref[...] = jnp.sum(jnp.exp(x_ref[...]), axis=1, keepdims=True)


def _tc_sumexp(x):
    return pl.pallas_call(
        _tc_sumexp_body,
        grid=(_R_TC // _RB,),
        in_specs=[pl.BlockSpec((_RB, _V), lambda r: (r, 0))],
        out_specs=pl.BlockSpec((_RB, 1), lambda r: (r, 0)),
        out_shape=jax.ShapeDtypeStruct((_R_TC, 1), jnp.float32),
    )(x)


_sc_mesh = plsc.VectorSubcoreMesh(core_axis_name="c", subcore_axis_name="s")


@functools.partial(
    pl.kernel,
    mesh=_sc_mesh,
    out_type=jax.ShapeDtypeStruct((_R_SC * _LANES,), jnp.float32),
    scratch_types=[
        pltpu.VMEM((8, _CW), jnp.float32),
        pltpu.VMEM((8, _CW), jnp.float32),
        pltpu.VMEM((8, _CTAIL), jnp.float32),
        pltpu.VMEM((_LANES,), jnp.float32),
        pltpu.SemaphoreType.DMA,
        pltpu.SemaphoreType.DMA,
        pltpu.SemaphoreType.DMA,
    ],
)
def _sc_sumexp(x_hbm, out_hbm, buf0, buf1, buft, rowbuf, sem0, sem1, semt):
    wid = lax.axis_index("s") * _NC + lax.axis_index("c")
    bufs = (buf0, buf1)
    sems = (sem0, sem1)

    def group_body(gi, carry):
        grp = wid * _GPWK + gi
        row0 = pl.multiple_of(_R_TC + grp * 8, 8)
        cps = [
            pltpu.make_async_copy(
                x_hbm.at[pl.ds(row0, 8), pl.ds(c * _CW, _CW)],
                bufs[c % 2],
                sems[c % 2],
            )
            for c in range(_NFULL)
        ]
        cpt = pltpu.make_async_copy(
            x_hbm.at[pl.ds(row0, 8), pl.ds(_NFULL * _CW, _CTAIL)],
            buft, semt)
        cps[0].start()
        accs = tuple(jnp.zeros((_LANES,), jnp.float32) for _ in range(8))
        for c in range(_NFULL):
            if c + 1 < _NFULL:
                cps[c + 1].start()
            else:
                cpt.start()
            cps[c].wait()
            buf = bufs[c % 2]

            def vbody(i, a):
                off = pl.multiple_of(i * _LANES, _LANES)
                return tuple(
                    a[r] + jnp.exp(buf[r, pl.ds(off, _LANES)])
                    for r in range(8)
                )

            accs = lax.fori_loop(0, _CW // _LANES, vbody, accs, unroll=2)
        cpt.wait()

        def vtail(i, a):
            off = pl.multiple_of(i * _LANES, _LANES)
            return tuple(
                a[r] + jnp.exp(buft[r, pl.ds(off, _LANES)])
                for r in range(8)
            )

        accs = lax.fori_loop(0, _CTAIL // _LANES, vtail, accs, unroll=2)
        for r in range(8):
            rowbuf[...] = accs[r]
            pltpu.sync_copy(
                rowbuf,
                out_hbm.at[pl.ds((grp * 8 + r) * _LANES, _LANES)])
        return carry

    lax.fori_loop(0, _GPWK, group_body, 0)


def _combine_body(g1_ref, g2_ref, gn_ref, anc_ref, stc_ref, psc_ref,
                  inst_ref, ans_ref):
    s_sc = jnp.sum(psc_ref[...], axis=1, keepdims=True)
    s = jnp.concatenate([stc_ref[...], s_sc], axis=0)
    s1 = s[:_HALF]
    s2 = s[_HALF:]
    p1 = jnp.exp(g1_ref[...]) / s1
    p2 = jnp.exp(g2_ref[...]) / s2
    pn = jnp.sum(jnp.exp(gn_ref[...]), axis=1, keepdims=True) / s1
    a = anc_ref[...]
    ans = -jnp.log(p1 + p2 + pn)
    inst = -jnp.log(p1 + p2)
    ans_ref[...] = (jnp.sum(a * ans) / _B).reshape(1, 1)
    inst_ref[...] = (jnp.sum((1.0 - a) * inst) / _B).reshape(1, 1)


def _combine(g1, g2, gn, anc, s_tc, partials_sc):
    return pl.pallas_call(
        _combine_body,
        out_shape=(
            jax.ShapeDtypeStruct((1, 1), jnp.float32),
            jax.ShapeDtypeStruct((1, 1), jnp.float32),
        ),
    )(g1, g2, gn, anc, s_tc, partials_sc)


def kernel(x, y, position, neighbours):
    s_tc = _tc_sumexp(x)
    partials_sc = _sc_sumexp(x).reshape(_R_SC, _LANES)

    y1 = y[:_HALF]
    y2 = y[_HALF:]
    pos = position[y1]
    anchor = (pos >= 0).astype(jnp.float32).reshape(_HALF, 1)
    pc = jnp.maximum(pos, 0)
    ncols = neighbours[pc]
    rows = jnp.arange(_HALF)
    g1 = x[rows, y1].reshape(_HALF, 1)
    g2 = x[rows + _HALF, y2].reshape(_HALF, 1)
    gn = x[rows[:, None], ncols]

    inst, ans = _combine(g1, g2, gn, anchor, s_tc, partials_sc)
    return (inst[0, 0], ans[0, 0])
